# baseline (speedup 1.0000x reference)
import functools

import jax
import jax.numpy as jnp
from jax import lax
from jax.experimental import pallas as pl
from jax.experimental.pallas import tpu as pltpu
from jax.experimental.pallas import tpu_sc as plsc

B = 4096
V = 1000
D = 10000
DA = 9984
WL = 5120
WR = DA - WL
NC, NS = 2, 16
NW = NC * NS
BPW = B // NW
C = 8
NCH = BPW // C


def _gather_grid(table, tail_t, idx3):
    mesh = plsc.VectorSubcoreMesh(core_axis_name="c", subcore_axis_name="s")

    @functools.partial(
        pl.kernel,
        out_type=jax.ShapeDtypeStruct((B, D), jnp.float32),
        mesh=mesh,
        compiler_params=pltpu.CompilerParams(use_tc_tiling_on_sc=True),
        scratch_types=[
            pltpu.VMEM((NCH, C), jnp.int32),
            pltpu.VMEM((C, WL), jnp.float32),
            pltpu.VMEM((C, WR), jnp.float32),
            pltpu.VMEM((C, 128), jnp.float32),
            pltpu.VMEM((C, 16), jnp.float32),
            pltpu.SemaphoreType.DMA,
            pltpu.SemaphoreType.DMA,
            pltpu.SemaphoreType.DMA,
            pltpu.SemaphoreType.DMA,
        ],
    )
    def k(table_hbm, tail_hbm, idx_hbm, out_hbm, idx_v, bufL, bufR, tailg,
          tailbuf, gsemL, gsemR, osemL, osemR):
        wid = lax.axis_index("s") * NC + lax.axis_index("c")
        base = wid * BPW
        pltpu.sync_copy(idx_hbm.at[wid], idx_v)

        def gatherL(c):
            return pltpu.make_async_copy(
                table_hbm.at[idx_v.at[c], pl.ds(0, WL)], bufL, gsemL
            )

        def gatherR(c):
            return pltpu.make_async_copy(
                table_hbm.at[idx_v.at[c], pl.ds(WL, WR)], bufR, gsemR
            )

        def gatherT(c):
            return pltpu.make_async_copy(
                tail_hbm.at[idx_v.at[c]], tailg, gsemR
            )

        def wbL(c):
            return pltpu.make_async_copy(
                bufL, out_hbm.at[pl.ds(base + c * C, C), pl.ds(0, WL)], osemL
            )

        def wbR(c):
            return pltpu.make_async_copy(
                bufR, out_hbm.at[pl.ds(base + c * C, C), pl.ds(WL, WR)], osemR
            )

        def wbTail(c):
            return pltpu.make_async_copy(
                tailbuf,
                out_hbm.at[pl.ds(base + c * C, C), pl.ds(DA, 16)],
                osemR,
            )

        def body(c, last):
            gatherR(c).start()
            gatherT(c).start()
            gatherL(c).wait()
            wbL(c).start()
            gatherR(c).wait()
            gatherT(c).wait()
            wbR(c).start()
            for r in range(C):
                tailbuf[r, :] = tailg[r, pl.ds(0, 16)]
            wbTail(c).start()
            if not last:
                wbL(c).wait()
                gatherL(c + 1).start()
            wbR(c).wait()
            wbTail(c).wait()

        gatherL(0).start()

        @pl.loop(0, NCH - 1)
        def _(c):
            body(c, last=False)

        body(NCH - 1, last=True)
        wbL(NCH - 1).wait()

    return k(table, tail_t, idx3)


def kernel(tokens, hdc_vocab):
    tail_t = jnp.pad(hdc_vocab[:, DA:], ((0, 0), (0, 128 - (D - DA))))
    idx3 = tokens.astype(jnp.int32).reshape(NW, NCH, C)
    return _gather_grid(hdc_vocab, tail_t, idx3)


# speedup vs baseline: 1.0160x; 1.0160x over previous
import functools

import jax
import jax.numpy as jnp
from jax import lax
from jax.experimental import pallas as pl
from jax.experimental.pallas import tpu as pltpu
from jax.experimental.pallas import tpu_sc as plsc

B = 4096
V = 1000
D = 10000
DA = 9984
WL = 5120
WR = DA - WL
NC, NS = 2, 16
NW = NC * NS
BPW = B // NW
C = 8
NCH = BPW // C


def _gather_grid(table, tail_t, idx3):
    mesh = plsc.VectorSubcoreMesh(core_axis_name="c", subcore_axis_name="s")

    @functools.partial(
        pl.kernel,
        out_type=jax.ShapeDtypeStruct((B, D), jnp.float32),
        mesh=mesh,
        compiler_params=pltpu.CompilerParams(use_tc_tiling_on_sc=True),
        scratch_types=[
            pltpu.VMEM((BPW,), jnp.int32),
            pltpu.VMEM((C, WL), jnp.float32),
            pltpu.VMEM((C, WR), jnp.float32),
            pltpu.VMEM((C, 128), jnp.float32),
            pltpu.VMEM((C, 16), jnp.float32),
            pltpu.SemaphoreType.DMA,
            pltpu.SemaphoreType.DMA,
            pltpu.SemaphoreType.DMA,
            pltpu.SemaphoreType.DMA,
        ],
    )
    def k(table_hbm, tail_hbm, idx_hbm, out_hbm, idx_v, bufL, bufR, tailg,
          tailbuf, gsemL, gsemR, osemL, osemR):
        wid = lax.axis_index("s") * NC + lax.axis_index("c")
        base = wid * BPW
        pltpu.sync_copy(idx_hbm.at[pl.ds(base, BPW)], idx_v)

        def gatherL(c):
            return pltpu.make_async_copy(
                table_hbm.at[idx_v.at[pl.ds(c * C, C)], pl.ds(0, WL)],
                bufL, gsemL,
            )

        def gatherR(c):
            return pltpu.make_async_copy(
                table_hbm.at[idx_v.at[pl.ds(c * C, C)], pl.ds(WL, WR)],
                bufR, gsemR,
            )

        def gatherT(c):
            return pltpu.make_async_copy(
                tail_hbm.at[idx_v.at[pl.ds(c * C, C)]], tailg, gsemR
            )

        def wbL(c):
            return pltpu.make_async_copy(
                bufL, out_hbm.at[pl.ds(base + c * C, C), pl.ds(0, WL)], osemL
            )

        def wbR(c):
            return pltpu.make_async_copy(
                bufR, out_hbm.at[pl.ds(base + c * C, C), pl.ds(WL, WR)], osemR
            )

        def wbTail(c):
            return pltpu.make_async_copy(
                tailbuf,
                out_hbm.at[pl.ds(base + c * C, C), pl.ds(DA, 16)],
                osemR,
            )

        def body(c, last):
            gatherL(c).wait()
            wbL(c).start()
            gatherR(c).start()
            gatherT(c).start()
            gatherR(c).wait()
            gatherT(c).wait()
            wbR(c).start()
            for r in range(C):
                tailbuf[r, :] = tailg[r, pl.ds(0, 16)]
            wbTail(c).start()
            if not last:
                wbL(c).wait()
                gatherL(c + 1).start()
            wbR(c).wait()
            wbTail(c).wait()

        gatherL(0).start()

        @pl.loop(0, NCH - 1)
        def _(c):
            body(c, last=False)

        body(NCH - 1, last=True)
        wbL(NCH - 1).wait()

    return k(table, tail_t, idx3)


def kernel(tokens, hdc_vocab):
    tail_t = jnp.pad(hdc_vocab[:, DA:], ((0, 0), (0, 128 - (D - DA))))
    return _gather_grid(hdc_vocab, tail_t, tokens.astype(jnp.int32))


# speedup vs baseline: 1.0164x; 1.0003x over previous
import functools

import jax
import jax.numpy as jnp
from jax import lax
from jax.experimental import pallas as pl
from jax.experimental.pallas import tpu as pltpu
from jax.experimental.pallas import tpu_sc as plsc

B = 4096
V = 1000
D = 10000
DA = 9984
WL = 5120
WR = DA - WL
NC, NS = 2, 16
NW = NC * NS
BPW = B // NW
C = 8
NCH = BPW // C


def _gather_grid(table, tail_t, tokens_i32):
    mesh = plsc.VectorSubcoreMesh(core_axis_name="c", subcore_axis_name="s")

    @functools.partial(
        pl.kernel,
        out_type=jax.ShapeDtypeStruct((B, D), jnp.float32),
        mesh=mesh,
        compiler_params=pltpu.CompilerParams(use_tc_tiling_on_sc=True),
        scratch_types=[
            pltpu.VMEM((BPW,), jnp.int32),
            pltpu.VMEM((C, WL), jnp.float32),
            pltpu.VMEM((C, WR), jnp.float32),
            pltpu.VMEM((C, 128), jnp.float32),
            pltpu.VMEM((C, 16), jnp.float32),
            pltpu.SemaphoreType.DMA,
            pltpu.SemaphoreType.DMA,
            pltpu.SemaphoreType.DMA,
            pltpu.SemaphoreType.DMA,
        ],
    )
    def k(table_hbm, tail_hbm, idx_hbm, out_hbm, idx_v, bufL, bufR, tailg,
          tailbuf, gsemL, gsemR, osemL, osemR):
        wid = lax.axis_index("s") * NC + lax.axis_index("c")
        base = wid * BPW
        pltpu.sync_copy(idx_hbm.at[pl.ds(base, BPW)], idx_v)

        def gatherL(c):
            return pltpu.make_async_copy(
                table_hbm.at[idx_v.at[pl.ds(c * C, C)], pl.ds(0, WL)],
                bufL, gsemL,
            )

        def gatherR(c):
            return pltpu.make_async_copy(
                table_hbm.at[idx_v.at[pl.ds(c * C, C)], pl.ds(WL, WR)],
                bufR, gsemR,
            )

        def gatherT(c):
            return pltpu.make_async_copy(
                tail_hbm.at[idx_v.at[pl.ds(c * C, C)]], tailg, gsemR
            )

        def wbL(c):
            return pltpu.make_async_copy(
                bufL, out_hbm.at[pl.ds(base + c * C, C), pl.ds(0, WL)], osemL
            )

        def wbR(c):
            return pltpu.make_async_copy(
                bufR, out_hbm.at[pl.ds(base + c * C, C), pl.ds(WL, WR)], osemR
            )

        def wbTail(c):
            return pltpu.make_async_copy(
                tailbuf,
                out_hbm.at[pl.ds(base + c * C, C), pl.ds(DA, 16)],
                osemR,
            )

        def body(c, last):
            gatherL(c).wait()
            wbL(c).start()
            gatherR(c).start()
            gatherT(c).start()
            gatherR(c).wait()
            gatherT(c).wait()
            wbR(c).start()
            for r in range(C):
                tailbuf[r, :] = tailg[r, pl.ds(0, 16)]
            wbTail(c).start()
            if not last:
                wbL(c).wait()
                gatherL(c + 1).start()
            wbR(c).wait()
            wbTail(c).wait()

        gatherL(0).start()

        @pl.loop(0, NCH - 1)
        def _(c):
            body(c, last=False)

        body(NCH - 1, last=True)
        wbL(NCH - 1).wait()

    return k(table, tail_t, tokens_i32)


def kernel(tokens, hdc_vocab):
    tail_t = jnp.pad(hdc_vocab[:, DA:], ((0, 0), (0, 128 - (D - DA))))
    return _gather_grid(hdc_vocab, tail_t, tokens.astype(jnp.int32))
